# Initial kernel scaffold; baseline (speedup 1.0000x reference)
import jax
import jax.numpy as jnp
from jax.experimental import pallas as pl


def kernel(x, edge_index, T, W_l1, b_l1, W_r1, W_l2, b_l2, W_r2):
    raise NotImplementedError("write your pallas kernel here")


# speedup vs baseline: 3.6021x; 3.6021x over previous
import functools

import jax
import jax.numpy as jnp
from jax import lax
from jax.experimental import pallas as pl
from jax.experimental.pallas import tpu as pltpu
from jax.experimental.pallas import tpu_sc as plsc

N = 10000
E = 160000
D = 256
H = 512
C = 64

NC = 2
NS = 16
NPAD = 10240
ROWS_PER_TILE = NPAD // NS

_SELU_ALPHA = 1.6732632423543772
_SELU_SCALE = 1.0507009873554805


def _make_sc_segsum(nb, fb, const_rows=False):
  split_edges = (nb == 1)
  blocks_per_core = 1 if split_edges else nb // 2
  epw = E // (NC * NS) if split_edges else E // NS
  K = 40 if split_edges else 80
  nchunks = epw // K
  assert nchunks * K == epw

  mesh = plsc.VectorSubcoreMesh(core_axis_name="c", subcore_axis_name="s")
  out_rows = 2 * NPAD if split_edges else nb * NPAD

  scratch = [
      pltpu.VMEM((K,), jnp.int32),
      pltpu.VMEM((K,), jnp.int32),
      pltpu.VMEM((K, fb), jnp.float32),
      pltpu.VMEM_SHARED((NPAD, fb), jnp.float32),
  ]

  @functools.partial(
      pl.kernel, mesh=mesh,
      out_type=jax.ShapeDtypeStruct((out_rows, fb), jnp.float32),
      scratch_types=scratch,
  )
  def k(h_hbm, src_hbm, dst_hbm, zeros_hbm, ones_hbm, out_hbm,
        sidx, didx, rows, acc):
    c = lax.axis_index("c")
    s = lax.axis_index("s")
    if split_edges:
      ebase = (s * NC + c) * epw
    else:
      ebase = s * epw
    r0 = s * ROWS_PER_TILE

    if const_rows:
      pltpu.sync_copy(ones_hbm, rows)

    for kb in range(blocks_per_core):
      bglob = c * blocks_per_core + kb if not split_edges else 0
      pltpu.sync_copy(zeros_hbm, acc.at[pl.ds(r0, ROWS_PER_TILE)])
      plsc.subcore_barrier()

      goff = bglob * N

      def body(i, carry):
        base = ebase + i * K
        pltpu.sync_copy(dst_hbm.at[pl.ds(base, K)], didx)
        if not const_rows:
          pltpu.sync_copy(src_hbm.at[pl.ds(base, K)], sidx)
          if not split_edges:
            for j in range(K // 16):
              sl = pl.ds(j * 16, 16)
              sidx[sl] = sidx[sl] + goff
          pltpu.sync_copy(h_hbm.at[sidx], rows)
        pltpu.sync_copy(rows, acc.at[didx], add=True)
        return carry

      lax.fori_loop(0, nchunks, body, 0)
      plsc.subcore_barrier()

      obase = (c * NPAD if split_edges else bglob * NPAD) + r0
      pltpu.sync_copy(acc.at[pl.ds(r0, ROWS_PER_TILE)],
                      out_hbm.at[pl.ds(obase, ROWS_PER_TILE)])
      if kb + 1 < blocks_per_core:
        plsc.subcore_barrier()

  return k


_segsum_d = _make_sc_segsum(D // 128, 128)
_segsum_h = _make_sc_segsum(H // 128, 128)
_segsum_c = _make_sc_segsum(1, 128)
_deg_hist = _make_sc_segsum(1, 128, const_rows=True)


R = 1000
GRID = N // R


def _deg_vec(parts_ref):
  d = parts_ref[0, :, :1] + parts_ref[1, :, :1]
  return d


def _dinv(deg):
  return jnp.where(deg > 0, lax.rsqrt(jnp.maximum(deg, 1e-12)), 0.0)


def _scale1_body(parts_ref, x_ref, out_ref):
  deg = _deg_vec(parts_ref)
  xs = x_ref[...] * _dinv(deg)
  for kk in range(2):
    out_ref[kk] = xs[:, kk * 128:(kk + 1) * 128]


def _scale2_body(parts_ref, y_ref, out_ref):
  deg = _deg_vec(parts_ref)
  dinv2 = jnp.where(deg > 0, 1.0 / jnp.maximum(deg, 1e-12), 0.0)
  for kk in range(2):
    out_ref[kk] = y_ref[kk] * dinv2


def _scale3_body(parts_ref, y_ref, blk_ref, flat_ref):
  deg = _deg_vec(parts_ref)
  di = _dinv(deg)
  hs = [y_ref[kk] * di for kk in range(2)]
  for kk in range(2):
    blk_ref[kk] = hs[kk]
  flat_ref[...] = jnp.concatenate(hs, axis=1)


def _selu(x):
  return _SELU_SCALE * jnp.where(x > 0, x, _SELU_ALPHA * (jnp.exp(x) - 1.0))


def _layer1_body(parts_ref, sh_ref, h_ref, wl_ref, bl_ref, wr_ref,
                 blk_ref, flat_ref):
  deg = _deg_vec(parts_ref)
  cnt = jnp.maximum(deg, 1.0)
  mean = jnp.concatenate([sh_ref[0], sh_ref[1]], axis=1) / cnt
  h1 = (jnp.dot(mean, wl_ref[...], preferred_element_type=jnp.float32)
        + bl_ref[...]
        + jnp.dot(h_ref[...], wr_ref[...], preferred_element_type=jnp.float32))
  h1 = _selu(h1)
  for kk in range(4):
    blk_ref[kk] = h1[:, kk * 128:(kk + 1) * 128]
  flat_ref[...] = h1


def _softmax(z):
  m = jnp.max(z, axis=1, keepdims=True)
  e = jnp.exp(z - m)
  return e / jnp.sum(e, axis=1, keepdims=True)


def _layer2_body(parts_ref, s1_ref, h1_ref, wl_ref, bl_ref, wr_ref, t_ref,
                 p_ref, q_ref, qs_ref):
  deg = _deg_vec(parts_ref)
  cnt = jnp.maximum(deg, 1.0)
  mean = jnp.concatenate([s1_ref[kk] for kk in range(4)], axis=1) / cnt
  h2 = (jnp.dot(mean, wl_ref[...], preferred_element_type=jnp.float32)
        + bl_ref[...]
        + jnp.dot(h1_ref[...], wr_ref[...], preferred_element_type=jnp.float32))
  p = _softmax(h2)
  q = jnp.dot(p, t_ref[...], preferred_element_type=jnp.float32)
  p_ref[...] = p
  q_ref[...] = q
  qs_ref[...] = jnp.concatenate(
      [q * _dinv(deg), jnp.zeros((q.shape[0], 128 - C), q.dtype)], axis=1)


def _scale4_body(parts_ref, y_ref, out_ref):
  deg = _deg_vec(parts_ref)
  dinv2 = jnp.where(deg > 0, 1.0 / jnp.maximum(deg, 1e-12), 0.0)
  out_ref[...] = (y_ref[0] + y_ref[1]) * dinv2


def _final_body(parts_ref, y_ref, out_ref):
  deg = _deg_vec(parts_ref)
  y = (y_ref[0] + y_ref[1])[:, :C] * _dinv(deg)
  out_ref[...] = _softmax(y)


def _bs(shape, imap):
  return pl.BlockSpec(shape, imap)


_PARTS_BS = _bs((2, R, 16), lambda i: (0, i, 0))


def _tc_call(body, in_specs, out_specs, out_shapes, *args):
  return pl.pallas_call(
      body, grid=(GRID,), in_specs=in_specs, out_specs=out_specs,
      out_shape=out_shapes)(*args)


def kernel(x, edge_index, T, W_l1, b_l1, W_r1, W_l2, b_l2, W_r2):
  src = edge_index[0]
  dst = edge_index[1]

  z128 = jnp.zeros((ROWS_PER_TILE, 128), jnp.float32)
  ones128 = jnp.ones((40, 128), jnp.float32)
  dummy80_128 = jnp.zeros((80, 128), jnp.float32)
  dummy40_128 = jnp.zeros((40, 128), jnp.float32)

  deg_parts = _deg_hist(jnp.zeros((8, 128), jnp.float32), src, dst,
                        z128, ones128)
  deg_parts = deg_parts.reshape(2, NPAD, 128)[:, :N, :16]

  xs_blk = _tc_call(
      _scale1_body,
      [_PARTS_BS, _bs((R, D), lambda i: (i, 0))],
      _bs((2, R, 128), lambda i: (0, i, 0)),
      jax.ShapeDtypeStruct((2, N, 128), jnp.float32),
      deg_parts, x)

  x2_blk = _segsum_d(xs_blk.reshape(2 * N, 128), src, dst, z128, dummy80_128)
  x2_blk = x2_blk.reshape(2, NPAD, 128)[:, :N]

  x2s_blk = _tc_call(
      _scale2_body,
      [_PARTS_BS, _bs((2, R, 128), lambda i: (0, i, 0))],
      _bs((2, R, 128), lambda i: (0, i, 0)),
      jax.ShapeDtypeStruct((2, N, 128), jnp.float32),
      deg_parts, x2_blk)

  x3_blk = _segsum_d(x2s_blk.reshape(2 * N, 128), src, dst, z128, dummy80_128)
  x3_blk = x3_blk.reshape(2, NPAD, 128)[:, :N]

  h_blk, h_flat = _tc_call(
      _scale3_body,
      [_PARTS_BS, _bs((2, R, 128), lambda i: (0, i, 0))],
      [_bs((2, R, 128), lambda i: (0, i, 0)), _bs((R, D), lambda i: (i, 0))],
      [jax.ShapeDtypeStruct((2, N, 128), jnp.float32),
       jax.ShapeDtypeStruct((N, D), jnp.float32)],
      deg_parts, x3_blk)

  sh_blk = _segsum_d(h_blk.reshape(2 * N, 128), src, dst, z128, dummy80_128)
  sh_blk = sh_blk.reshape(2, NPAD, 128)[:, :N]

  b_l1r = b_l1.reshape(1, H)
  h1_blk, h1_flat = _tc_call(
      _layer1_body,
      [_PARTS_BS,
       _bs((2, R, 128), lambda i: (0, i, 0)),
       _bs((R, D), lambda i: (i, 0)),
       _bs((D, H), lambda i: (0, 0)),
       _bs((1, H), lambda i: (0, 0)),
       _bs((D, H), lambda i: (0, 0))],
      [_bs((4, R, 128), lambda i: (0, i, 0)), _bs((R, H), lambda i: (i, 0))],
      [jax.ShapeDtypeStruct((4, N, 128), jnp.float32),
       jax.ShapeDtypeStruct((N, H), jnp.float32)],
      deg_parts, sh_blk, h_flat, W_l1, b_l1r, W_r1)

  s1_blk = _segsum_h(h1_blk.reshape(4 * N, 128), src, dst, z128, dummy80_128)
  s1_blk = s1_blk.reshape(4, NPAD, 128)[:, :N]

  b_l2r = b_l2.reshape(1, C)
  p, q, qs = _tc_call(
      _layer2_body,
      [_PARTS_BS,
       _bs((4, R, 128), lambda i: (0, i, 0)),
       _bs((R, H), lambda i: (i, 0)),
       _bs((H, C), lambda i: (0, 0)),
       _bs((1, C), lambda i: (0, 0)),
       _bs((H, C), lambda i: (0, 0)),
       _bs((C, C), lambda i: (0, 0))],
      [_bs((R, C), lambda i: (i, 0))] * 2 + [_bs((R, 128), lambda i: (i, 0))],
      [jax.ShapeDtypeStruct((N, C), jnp.float32)] * 2
      + [jax.ShapeDtypeStruct((N, 128), jnp.float32)],
      deg_parts, s1_blk, h1_flat, W_l2, b_l2r, W_r2, T)

  y1_parts = _segsum_c(qs, src, dst, z128,
                       dummy40_128).reshape(2, NPAD, 128)[:, :N]

  y1s = _tc_call(
      _scale4_body,
      [_PARTS_BS, _bs((2, R, 128), lambda i: (0, i, 0))],
      _bs((R, 128), lambda i: (i, 0)),
      jax.ShapeDtypeStruct((N, 128), jnp.float32),
      deg_parts, y1_parts)

  y2_parts = _segsum_c(y1s, src, dst, z128,
                       dummy40_128).reshape(2, NPAD, 128)[:, :N]

  p_yt = _tc_call(
      _final_body,
      [_PARTS_BS, _bs((2, R, 128), lambda i: (0, i, 0))],
      _bs((R, C), lambda i: (i, 0)),
      jax.ShapeDtypeStruct((N, C), jnp.float32),
      deg_parts, y2_parts)

  return (p, q, p_yt)


# speedup vs baseline: 8.4715x; 2.3519x over previous
import functools

import jax
import jax.numpy as jnp
from jax import lax
from jax.experimental import pallas as pl
from jax.experimental.pallas import tpu as pltpu
from jax.experimental.pallas import tpu_sc as plsc

N = 10000
E = 160000
D = 256
H = 512
C = 64

NC = 2
NS = 16
NPAD = 10240
ROWS_PER_TILE = NPAD // NS

_SELU_ALPHA = 1.6732632423543772
_SELU_SCALE = 1.0507009873554805


NBUF = 5
PRE = 3


def _make_sc_segsum(nb, fb, const_rows=False):
  split_edges = (nb == 1)
  blocks_per_core = 1 if split_edges else nb // 2
  epw = E // (NC * NS) if split_edges else E // NS
  K = 40
  nchunks = epw // K
  GRP = 50 if not split_edges else 25
  assert nchunks % GRP == 0 and GRP % NBUF == 0
  ngroups = nchunks // GRP

  mesh = plsc.VectorSubcoreMesh(core_axis_name="c", subcore_axis_name="s")
  out_rows = 2 * NPAD if split_edges else nb * NPAD

  scratch = (
      [pltpu.VMEM((GRP * K,), jnp.int32),
       pltpu.VMEM((GRP, 1, K), jnp.int32)]
      + [pltpu.VMEM((K, fb), jnp.float32) for _ in range(NBUF)]
      + [pltpu.SemaphoreType.DMA for _ in range(2 * NBUF)]
      + [pltpu.VMEM_SHARED((NPAD, fb), jnp.float32)]
  )

  @functools.partial(
      pl.kernel, mesh=mesh,
      out_type=jax.ShapeDtypeStruct((out_rows, fb), jnp.float32),
      scratch_types=scratch,
  )
  def k(h_hbm, srcx_hbm, dstx_hbm, zeros_hbm, ones_hbm, out_hbm,
        sidx_g, didx_g, *bufs_sems_acc):
    rows = list(bufs_sems_acc[:NBUF])
    gsem = list(bufs_sems_acc[NBUF:2 * NBUF])
    ssem = list(bufs_sems_acc[2 * NBUF:3 * NBUF])
    acc = bufs_sems_acc[3 * NBUF]

    c = lax.axis_index("c")
    s = lax.axis_index("s")
    wid = s * NC + c
    ebase = (wid if split_edges else s) * epw
    cbase = (wid if split_edges else s) * nchunks
    r0 = s * ROWS_PER_TILE

    if const_rows:
      pltpu.sync_copy(ones_hbm, rows[0])

    def gather(l, b):
      pltpu.async_copy(h_hbm.at[sidx_g.at[pl.ds(l * K, K)]], rows[b], gsem[b])

    def gwait(b):
      pltpu.make_async_copy(h_hbm.at[sidx_g.at[pl.ds(0, K)]], rows[b],
                            gsem[b]).wait()

    def scatter(l, b):
      pltpu.async_copy(rows[0 if const_rows else b],
                       acc.at[didx_g.at[l, 0]], ssem[b], add=True)

    def swait(b):
      pltpu.make_async_copy(rows[0 if const_rows else b],
                            acc.at[didx_g.at[0, 0]], ssem[b]).wait()

    for kb in range(blocks_per_core):
      bglob = 0 if split_edges else c * blocks_per_core + kb
      pltpu.sync_copy(zeros_hbm, acc.at[pl.ds(r0, ROWS_PER_TILE)])
      plsc.subcore_barrier()

      def group(g, carry):
        pltpu.sync_copy(dstx_hbm.at[pl.ds(cbase + g * GRP, GRP)], didx_g)
        if const_rows:
          def cbody(t, carry2):
            for b in range(NBUF):
              l = t * NBUF + b
              pl.when(t > 0)(functools.partial(swait, b))
              scatter(l, b)
            return carry2

          lax.fori_loop(0, GRP // NBUF, cbody, 0)
        else:
          pltpu.sync_copy(
              srcx_hbm.at[pl.ds(bglob * E + ebase + g * GRP * K, GRP * K)],
              sidx_g)
          for b in range(PRE):
            gather(b, b)

          def body(t, carry2):
            for b in range(NBUF):
              l = t * NBUF + b
              gwait(b)
              scatter(l, b)
              bn = (b + PRE) % NBUF

              def refill(bn=bn, l=l):
                pl.when(l >= NBUF - PRE)(functools.partial(swait, bn))
                gather(l + PRE, bn)

              pl.when(l + PRE < GRP)(refill)
            return carry2

          lax.fori_loop(0, GRP // NBUF, body, 0)
        for b in range(NBUF):
          swait(b)
        return carry

      lax.fori_loop(0, ngroups, group, 0)

      plsc.subcore_barrier()
      obase = (c * NPAD if split_edges else bglob * NPAD) + r0
      pltpu.sync_copy(acc.at[pl.ds(r0, ROWS_PER_TILE)],
                      out_hbm.at[pl.ds(obase, ROWS_PER_TILE)])
      if kb + 1 < blocks_per_core:
        plsc.subcore_barrier()

  return k


_segsum_d = _make_sc_segsum(D // 128, 128)
_segsum_h = _make_sc_segsum(H // 128, 128)
_segsum_c = _make_sc_segsum(1, 128)
_deg_hist = _make_sc_segsum(1, 128, const_rows=True)


R = 1000
GRID = N // R


def _deg_vec(parts_ref):
  d = parts_ref[0, :, :1] + parts_ref[1, :, :1]
  return d


def _dinv(deg):
  return jnp.where(deg > 0, lax.rsqrt(jnp.maximum(deg, 1e-12)), 0.0)


def _scale1_body(parts_ref, x_ref, out_ref):
  deg = _deg_vec(parts_ref)
  xs = x_ref[...] * _dinv(deg)
  for kk in range(2):
    out_ref[kk] = xs[:, kk * 128:(kk + 1) * 128]


def _scale2_body(parts_ref, y_ref, out_ref):
  deg = _deg_vec(parts_ref)
  dinv2 = jnp.where(deg > 0, 1.0 / jnp.maximum(deg, 1e-12), 0.0)
  for kk in range(2):
    out_ref[kk] = y_ref[kk] * dinv2


def _scale3_body(parts_ref, y_ref, blk_ref, flat_ref):
  deg = _deg_vec(parts_ref)
  di = _dinv(deg)
  hs = [y_ref[kk] * di for kk in range(2)]
  for kk in range(2):
    blk_ref[kk] = hs[kk]
  flat_ref[...] = jnp.concatenate(hs, axis=1)


def _selu(x):
  return _SELU_SCALE * jnp.where(x > 0, x, _SELU_ALPHA * (jnp.exp(x) - 1.0))


def _layer1_body(parts_ref, sh_ref, h_ref, wl_ref, bl_ref, wr_ref,
                 blk_ref, flat_ref):
  deg = _deg_vec(parts_ref)
  cnt = jnp.maximum(deg, 1.0)
  mean = jnp.concatenate([sh_ref[0], sh_ref[1]], axis=1) / cnt
  h1 = (jnp.dot(mean, wl_ref[...], preferred_element_type=jnp.float32)
        + bl_ref[...]
        + jnp.dot(h_ref[...], wr_ref[...], preferred_element_type=jnp.float32))
  h1 = _selu(h1)
  for kk in range(4):
    blk_ref[kk] = h1[:, kk * 128:(kk + 1) * 128]
  flat_ref[...] = h1


def _softmax(z):
  m = jnp.max(z, axis=1, keepdims=True)
  e = jnp.exp(z - m)
  return e / jnp.sum(e, axis=1, keepdims=True)


def _layer2_body(parts_ref, s1_ref, h1_ref, wl_ref, bl_ref, wr_ref, t_ref,
                 p_ref, q_ref, qs_ref):
  deg = _deg_vec(parts_ref)
  cnt = jnp.maximum(deg, 1.0)
  mean = jnp.concatenate([s1_ref[kk] for kk in range(4)], axis=1) / cnt
  h2 = (jnp.dot(mean, wl_ref[...], preferred_element_type=jnp.float32)
        + bl_ref[...]
        + jnp.dot(h1_ref[...], wr_ref[...], preferred_element_type=jnp.float32))
  p = _softmax(h2)
  q = jnp.dot(p, t_ref[...], preferred_element_type=jnp.float32)
  p_ref[...] = p
  q_ref[...] = q
  qs_ref[...] = jnp.concatenate(
      [q * _dinv(deg), jnp.zeros((q.shape[0], 128 - C), q.dtype)], axis=1)


def _scale4_body(parts_ref, y_ref, out_ref):
  deg = _deg_vec(parts_ref)
  dinv2 = jnp.where(deg > 0, 1.0 / jnp.maximum(deg, 1e-12), 0.0)
  out_ref[...] = (y_ref[0] + y_ref[1]) * dinv2


def _final_body(parts_ref, y_ref, out_ref):
  deg = _deg_vec(parts_ref)
  y = (y_ref[0] + y_ref[1])[:, :C] * _dinv(deg)
  out_ref[...] = _softmax(y)


def _bs(shape, imap):
  return pl.BlockSpec(shape, imap)


_PARTS_BS = _bs((2, R, 16), lambda i: (0, i, 0))


def _tc_call(body, in_specs, out_specs, out_shapes, *args):
  return pl.pallas_call(
      body, grid=(GRID,), in_specs=in_specs, out_specs=out_specs,
      out_shape=out_shapes)(*args)


def kernel(x, edge_index, T, W_l1, b_l1, W_r1, W_l2, b_l2, W_r2):
  src = edge_index[0]
  dst = edge_index[1]

  z128 = jnp.zeros((ROWS_PER_TILE, 128), jnp.float32)
  ones128 = jnp.ones((40, 128), jnp.float32)
  d40 = jnp.zeros((40, 128), jnp.float32)

  dstx40 = dst.reshape(E // 40, 1, 40)
  srcx1 = src
  off2 = (jnp.arange(2, dtype=jnp.int32) * N)[:, None]
  srcx2 = (src[None, :] + off2).reshape(2 * E)
  off4 = (jnp.arange(4, dtype=jnp.int32) * N)[:, None]
  srcx4 = (src[None, :] + off4).reshape(4 * E)

  deg_parts = _deg_hist(jnp.zeros((8, 128), jnp.float32), srcx1, dstx40,
                        z128, ones128)
  deg_parts = deg_parts.reshape(2, NPAD, 128)[:, :N, :16]

  xs_blk = _tc_call(
      _scale1_body,
      [_PARTS_BS, _bs((R, D), lambda i: (i, 0))],
      _bs((2, R, 128), lambda i: (0, i, 0)),
      jax.ShapeDtypeStruct((2, N, 128), jnp.float32),
      deg_parts, x)

  x2_blk = _segsum_d(xs_blk.reshape(2 * N, 128), srcx2, dstx40, z128, d40)
  x2_blk = x2_blk.reshape(2, NPAD, 128)[:, :N]

  x2s_blk = _tc_call(
      _scale2_body,
      [_PARTS_BS, _bs((2, R, 128), lambda i: (0, i, 0))],
      _bs((2, R, 128), lambda i: (0, i, 0)),
      jax.ShapeDtypeStruct((2, N, 128), jnp.float32),
      deg_parts, x2_blk)

  x3_blk = _segsum_d(x2s_blk.reshape(2 * N, 128), srcx2, dstx40, z128, d40)
  x3_blk = x3_blk.reshape(2, NPAD, 128)[:, :N]

  h_blk, h_flat = _tc_call(
      _scale3_body,
      [_PARTS_BS, _bs((2, R, 128), lambda i: (0, i, 0))],
      [_bs((2, R, 128), lambda i: (0, i, 0)), _bs((R, D), lambda i: (i, 0))],
      [jax.ShapeDtypeStruct((2, N, 128), jnp.float32),
       jax.ShapeDtypeStruct((N, D), jnp.float32)],
      deg_parts, x3_blk)

  sh_blk = _segsum_d(h_blk.reshape(2 * N, 128), srcx2, dstx40, z128, d40)
  sh_blk = sh_blk.reshape(2, NPAD, 128)[:, :N]

  b_l1r = b_l1.reshape(1, H)
  h1_blk, h1_flat = _tc_call(
      _layer1_body,
      [_PARTS_BS,
       _bs((2, R, 128), lambda i: (0, i, 0)),
       _bs((R, D), lambda i: (i, 0)),
       _bs((D, H), lambda i: (0, 0)),
       _bs((1, H), lambda i: (0, 0)),
       _bs((D, H), lambda i: (0, 0))],
      [_bs((4, R, 128), lambda i: (0, i, 0)), _bs((R, H), lambda i: (i, 0))],
      [jax.ShapeDtypeStruct((4, N, 128), jnp.float32),
       jax.ShapeDtypeStruct((N, H), jnp.float32)],
      deg_parts, sh_blk, h_flat, W_l1, b_l1r, W_r1)

  s1_blk = _segsum_h(h1_blk.reshape(4 * N, 128), srcx4, dstx40, z128, d40)
  s1_blk = s1_blk.reshape(4, NPAD, 128)[:, :N]

  b_l2r = b_l2.reshape(1, C)
  p, q, qs = _tc_call(
      _layer2_body,
      [_PARTS_BS,
       _bs((4, R, 128), lambda i: (0, i, 0)),
       _bs((R, H), lambda i: (i, 0)),
       _bs((H, C), lambda i: (0, 0)),
       _bs((1, C), lambda i: (0, 0)),
       _bs((H, C), lambda i: (0, 0)),
       _bs((C, C), lambda i: (0, 0))],
      [_bs((R, C), lambda i: (i, 0))] * 2 + [_bs((R, 128), lambda i: (i, 0))],
      [jax.ShapeDtypeStruct((N, C), jnp.float32)] * 2
      + [jax.ShapeDtypeStruct((N, 128), jnp.float32)],
      deg_parts, s1_blk, h1_flat, W_l2, b_l2r, W_r2, T)

  y1_parts = _segsum_c(qs, srcx1, dstx40, z128,
                       d40).reshape(2, NPAD, 128)[:, :N]

  y1s = _tc_call(
      _scale4_body,
      [_PARTS_BS, _bs((2, R, 128), lambda i: (0, i, 0))],
      _bs((R, 128), lambda i: (i, 0)),
      jax.ShapeDtypeStruct((N, 128), jnp.float32),
      deg_parts, y1_parts)

  y2_parts = _segsum_c(y1s, srcx1, dstx40, z128,
                       d40).reshape(2, NPAD, 128)[:, :N]

  p_yt = _tc_call(
      _final_body,
      [_PARTS_BS, _bs((2, R, 128), lambda i: (0, i, 0))],
      _bs((R, C), lambda i: (i, 0)),
      jax.ShapeDtypeStruct((N, C), jnp.float32),
      deg_parts, y2_parts)

  return (p, q, p_yt)


# speedup vs baseline: 9.0097x; 1.0635x over previous
import functools

import jax
import jax.numpy as jnp
from jax import lax
from jax.experimental import pallas as pl
from jax.experimental.pallas import tpu as pltpu
from jax.experimental.pallas import tpu_sc as plsc

N = 10000
E = 160000
D = 256
H = 512
C = 64

NC = 2
NS = 16
NPAD = 10240
ROWS_PER_TILE = NPAD // NS

_SELU_ALPHA = 1.6732632423543772
_SELU_SCALE = 1.0507009873554805


NBUF = 5
PRE = 3


def _make_sc_segsum(nb, fb, const_rows=False):
  split_edges = (nb == 1)
  blocks_per_core = 1 if split_edges else nb // 2
  epw = E // (NC * NS) if split_edges else E // NS
  K = 40
  nchunks = epw // K
  GRP = 50 if not split_edges else 25
  assert nchunks % GRP == 0 and GRP % NBUF == 0
  ngroups = nchunks // GRP

  mesh = plsc.VectorSubcoreMesh(core_axis_name="c", subcore_axis_name="s")
  out_rows = 2 * NPAD if split_edges else nb * NPAD

  scratch = (
      [pltpu.VMEM((GRP * K,), jnp.int32),
       pltpu.VMEM((GRP, 1, K), jnp.int32)]
      + [pltpu.VMEM((K, fb), jnp.float32) for _ in range(NBUF)]
      + [pltpu.SemaphoreType.DMA for _ in range(2 * NBUF)]
      + [pltpu.VMEM_SHARED((NPAD, fb), jnp.float32)]
  )

  @functools.partial(
      pl.kernel, mesh=mesh,
      out_type=jax.ShapeDtypeStruct((out_rows, fb), jnp.float32),
      scratch_types=scratch,
  )
  def k(h_hbm, srcx_hbm, dstx_hbm, zeros_hbm, ones_hbm, out_hbm,
        sidx_g, didx_g, *bufs_sems_acc):
    rows = list(bufs_sems_acc[:NBUF])
    gsem = list(bufs_sems_acc[NBUF:2 * NBUF])
    ssem = list(bufs_sems_acc[2 * NBUF:3 * NBUF])
    acc = bufs_sems_acc[3 * NBUF]

    c = lax.axis_index("c")
    s = lax.axis_index("s")
    wid = s * NC + c
    ebase = (wid if split_edges else s) * epw
    cbase = (wid if split_edges else s) * nchunks
    r0 = s * ROWS_PER_TILE

    if const_rows:
      pltpu.sync_copy(ones_hbm, rows[0])

    def gather(l, b):
      pltpu.async_copy(h_hbm.at[sidx_g.at[pl.ds(l * K, K)]], rows[b], gsem[b])

    def gwait(b):
      pltpu.make_async_copy(h_hbm.at[sidx_g.at[pl.ds(0, K)]], rows[b],
                            gsem[b]).wait()

    def scatter(l, b):
      pltpu.async_copy(rows[0 if const_rows else b],
                       acc.at[didx_g.at[l, 0]], ssem[b], add=True)

    def swait(b):
      pltpu.make_async_copy(rows[0 if const_rows else b],
                            acc.at[didx_g.at[0, 0]], ssem[b]).wait()

    for kb in range(blocks_per_core):
      bglob = 0 if split_edges else c * blocks_per_core + kb
      pltpu.sync_copy(zeros_hbm, acc.at[pl.ds(r0, ROWS_PER_TILE)])
      plsc.subcore_barrier()

      def group(g, carry):
        pltpu.sync_copy(dstx_hbm.at[pl.ds(cbase + g * GRP, GRP)], didx_g)
        if const_rows:
          def cbody(t, carry2):
            for b in range(NBUF):
              l = t * NBUF + b
              pl.when(t > 0)(functools.partial(swait, b))
              scatter(l, b)
            return carry2

          lax.fori_loop(0, GRP // NBUF, cbody, 0)
        else:
          pltpu.sync_copy(
              srcx_hbm.at[pl.ds(bglob * E + ebase + g * GRP * K, GRP * K)],
              sidx_g)
          for b in range(PRE):
            gather(b, b)

          def body(t, carry2):
            for b in range(NBUF):
              l = t * NBUF + b
              gwait(b)
              scatter(l, b)
              bn = (b + PRE) % NBUF

              def refill(bn=bn, l=l):
                pl.when(l >= NBUF - PRE)(functools.partial(swait, bn))
                gather(l + PRE, bn)

              pl.when(l + PRE < GRP)(refill)
            return carry2

          lax.fori_loop(0, GRP // NBUF, body, 0)
        for b in range(NBUF):
          swait(b)
        return carry

      lax.fori_loop(0, ngroups, group, 0)

      plsc.subcore_barrier()
      obase = (c * NPAD if split_edges else bglob * NPAD) + r0
      pltpu.sync_copy(acc.at[pl.ds(r0, ROWS_PER_TILE)],
                      out_hbm.at[pl.ds(obase, ROWS_PER_TILE)])
      if kb + 1 < blocks_per_core:
        plsc.subcore_barrier()

  return k


_segsum_d = _make_sc_segsum(D // 128, 128)
_segsum_h = _make_sc_segsum(H // 128, 128)
_segsum_c = _make_sc_segsum(1, 128)
_deg_hist = _make_sc_segsum(1, 128, const_rows=True)


R = 1000
GRID = N // R


def _deg_vec(parts_ref):
  d = parts_ref[0, :, :1] + parts_ref[1, :, :1]
  return d


def _dinv(deg):
  return jnp.where(deg > 0, lax.rsqrt(jnp.maximum(deg, 1e-12)), 0.0)


def _scale1_body(parts_ref, x_ref, out_ref):
  deg = _deg_vec(parts_ref)
  xs = x_ref[...] * _dinv(deg)
  for kk in range(2):
    out_ref[kk] = xs[:, kk * 128:(kk + 1) * 128]


def _scale2_body(parts_ref, y_ref, out_ref):
  deg = _deg_vec(parts_ref)
  dinv2 = jnp.where(deg > 0, 1.0 / jnp.maximum(deg, 1e-12), 0.0)
  for kk in range(2):
    out_ref[kk] = y_ref[kk] * dinv2


def _scale3_body(parts_ref, y_ref, blk_ref, flat_ref):
  deg = _deg_vec(parts_ref)
  di = _dinv(deg)
  hs = [y_ref[kk] * di for kk in range(2)]
  for kk in range(2):
    blk_ref[kk] = hs[kk]
  flat_ref[...] = jnp.concatenate(hs, axis=1)


def _selu(x):
  return _SELU_SCALE * jnp.where(x > 0, x, _SELU_ALPHA * (jnp.exp(x) - 1.0))


def _layer1_body(parts_ref, sh_ref, h_ref, wl_ref, bl_ref, wr_ref,
                 blk_ref, flat_ref):
  deg = _deg_vec(parts_ref)
  cnt = jnp.maximum(deg, 1.0)
  mean = jnp.concatenate([sh_ref[0], sh_ref[1]], axis=1) / cnt
  h1 = (jnp.dot(mean, wl_ref[...], preferred_element_type=jnp.float32)
        + bl_ref[...]
        + jnp.dot(h_ref[...], wr_ref[...], preferred_element_type=jnp.float32))
  h1 = _selu(h1)
  for kk in range(4):
    blk_ref[kk] = h1[:, kk * 128:(kk + 1) * 128]
  flat_ref[...] = h1


def _softmax(z):
  m = jnp.max(z, axis=1, keepdims=True)
  e = jnp.exp(z - m)
  return e / jnp.sum(e, axis=1, keepdims=True)


def _layer2_body(parts_ref, s1_ref, h1_ref, wl_ref, bl_ref, wr_ref, t_ref,
                 p_ref, q_ref, qs_ref):
  deg = _deg_vec(parts_ref)
  cnt = jnp.maximum(deg, 1.0)
  mean = jnp.concatenate([s1_ref[kk] for kk in range(4)], axis=1) / cnt
  h2 = (jnp.dot(mean, wl_ref[...], preferred_element_type=jnp.float32)
        + bl_ref[...]
        + jnp.dot(h1_ref[...], wr_ref[...], preferred_element_type=jnp.float32))
  p = _softmax(h2)
  q = jnp.dot(p, t_ref[...], preferred_element_type=jnp.float32)
  p_ref[...] = p
  q_ref[...] = q
  qs_ref[...] = jnp.concatenate(
      [q * _dinv(deg), jnp.zeros((q.shape[0], 128 - C), q.dtype)], axis=1)


def _scale4_body(parts_ref, y_ref, out_ref):
  deg = _deg_vec(parts_ref)
  dinv2 = jnp.where(deg > 0, 1.0 / jnp.maximum(deg, 1e-12), 0.0)
  out_ref[...] = (y_ref[0] + y_ref[1]) * dinv2


def _final_body(parts_ref, y_ref, out_ref):
  deg = _deg_vec(parts_ref)
  y = (y_ref[0] + y_ref[1])[:, :C] * _dinv(deg)
  out_ref[...] = _softmax(y)


def _bs(shape, imap):
  return pl.BlockSpec(shape, imap)


_PARTS_BS = _bs((2, R, 16), lambda i: (0, i, 0))


def _tc_call(body, in_specs, out_specs, out_shapes, *args):
  return pl.pallas_call(
      body, grid=(GRID,), in_specs=in_specs, out_specs=out_specs,
      out_shape=out_shapes)(*args)


def kernel(x, edge_index, T, W_l1, b_l1, W_r1, W_l2, b_l2, W_r2):
  src = edge_index[0]
  dst = edge_index[1]

  z128 = jnp.zeros((ROWS_PER_TILE, 128), jnp.float32)
  ones128 = jnp.ones((40, 128), jnp.float32)
  d40 = jnp.zeros((40, 128), jnp.float32)

  dstx40 = dst.reshape(E // 40, 1, 40)
  srcx1 = src
  off2 = (jnp.arange(2, dtype=jnp.int32) * NPAD)[:, None]
  srcx2 = (src[None, :] + off2).reshape(2 * E)
  off4 = (jnp.arange(4, dtype=jnp.int32) * NPAD)[:, None]
  srcx4 = (src[None, :] + off4).reshape(4 * E)

  deg_parts = _deg_hist(jnp.zeros((8, 128), jnp.float32), srcx1, dstx40,
                        z128, ones128)
  deg_parts = deg_parts.reshape(2, NPAD, 128)[:, :, :16]

  xs_blk = _tc_call(
      _scale1_body,
      [_PARTS_BS, _bs((R, D), lambda i: (i, 0))],
      _bs((2, R, 128), lambda i: (0, i, 0)),
      jax.ShapeDtypeStruct((2, NPAD, 128), jnp.float32),
      deg_parts, x)

  x2_blk = _segsum_d(xs_blk.reshape(2 * NPAD, 128), srcx2, dstx40, z128, d40)
  x2_blk = x2_blk.reshape(2, NPAD, 128)

  x2s_blk = _tc_call(
      _scale2_body,
      [_PARTS_BS, _bs((2, R, 128), lambda i: (0, i, 0))],
      _bs((2, R, 128), lambda i: (0, i, 0)),
      jax.ShapeDtypeStruct((2, NPAD, 128), jnp.float32),
      deg_parts, x2_blk)

  x3_blk = _segsum_d(x2s_blk.reshape(2 * NPAD, 128), srcx2, dstx40, z128, d40)
  x3_blk = x3_blk.reshape(2, NPAD, 128)

  h_blk, h_flat = _tc_call(
      _scale3_body,
      [_PARTS_BS, _bs((2, R, 128), lambda i: (0, i, 0))],
      [_bs((2, R, 128), lambda i: (0, i, 0)), _bs((R, D), lambda i: (i, 0))],
      [jax.ShapeDtypeStruct((2, NPAD, 128), jnp.float32),
       jax.ShapeDtypeStruct((N, D), jnp.float32)],
      deg_parts, x3_blk)

  sh_blk = _segsum_d(h_blk.reshape(2 * NPAD, 128), srcx2, dstx40, z128, d40)
  sh_blk = sh_blk.reshape(2, NPAD, 128)

  b_l1r = b_l1.reshape(1, H)
  h1_blk, h1_flat = _tc_call(
      _layer1_body,
      [_PARTS_BS,
       _bs((2, R, 128), lambda i: (0, i, 0)),
       _bs((R, D), lambda i: (i, 0)),
       _bs((D, H), lambda i: (0, 0)),
       _bs((1, H), lambda i: (0, 0)),
       _bs((D, H), lambda i: (0, 0))],
      [_bs((4, R, 128), lambda i: (0, i, 0)), _bs((R, H), lambda i: (i, 0))],
      [jax.ShapeDtypeStruct((4, NPAD, 128), jnp.float32),
       jax.ShapeDtypeStruct((N, H), jnp.float32)],
      deg_parts, sh_blk, h_flat, W_l1, b_l1r, W_r1)

  s1_blk = _segsum_h(h1_blk.reshape(4 * NPAD, 128), srcx4, dstx40, z128, d40)
  s1_blk = s1_blk.reshape(4, NPAD, 128)

  b_l2r = b_l2.reshape(1, C)
  p, q, qs = _tc_call(
      _layer2_body,
      [_PARTS_BS,
       _bs((4, R, 128), lambda i: (0, i, 0)),
       _bs((R, H), lambda i: (i, 0)),
       _bs((H, C), lambda i: (0, 0)),
       _bs((1, C), lambda i: (0, 0)),
       _bs((H, C), lambda i: (0, 0)),
       _bs((C, C), lambda i: (0, 0))],
      [_bs((R, C), lambda i: (i, 0))] * 2 + [_bs((R, 128), lambda i: (i, 0))],
      [jax.ShapeDtypeStruct((N, C), jnp.float32)] * 2
      + [jax.ShapeDtypeStruct((N, 128), jnp.float32)],
      deg_parts, s1_blk, h1_flat, W_l2, b_l2r, W_r2, T)

  y1_parts = _segsum_c(qs, srcx1, dstx40, z128,
                       d40).reshape(2, NPAD, 128)

  y1s = _tc_call(
      _scale4_body,
      [_PARTS_BS, _bs((2, R, 128), lambda i: (0, i, 0))],
      _bs((R, 128), lambda i: (i, 0)),
      jax.ShapeDtypeStruct((N, 128), jnp.float32),
      deg_parts, y1_parts)

  y2_parts = _segsum_c(y1s, srcx1, dstx40, z128,
                       d40).reshape(2, NPAD, 128)

  p_yt = _tc_call(
      _final_body,
      [_PARTS_BS, _bs((2, R, 128), lambda i: (0, i, 0))],
      _bs((R, C), lambda i: (i, 0)),
      jax.ShapeDtypeStruct((N, C), jnp.float32),
      deg_parts, y2_parts)

  return (p, q, p_yt)


# speedup vs baseline: 11.0371x; 1.2250x over previous
import functools

import jax
import jax.numpy as jnp
from jax import lax
from jax.experimental import pallas as pl
from jax.experimental.pallas import tpu as pltpu
from jax.experimental.pallas import tpu_sc as plsc

N = 10000
E = 160000
D = 256
H = 512
C = 64

NC = 2
NS = 16
NPAD = 10240
ROWS_PER_TILE = NPAD // NS

_SELU_ALPHA = 1.6732632423543772
_SELU_SCALE = 1.0507009873554805


NBUF = 5
PRE = 3


def _make_sc_segsum(nb, fb, const_rows=False):
  split_edges = (nb == 1)
  blocks_per_core = 1 if split_edges else nb // 2
  epw = E // (NC * NS) if split_edges else E // NS
  K = 40
  nchunks = epw // K
  GRP = 50 if not split_edges else 25
  assert nchunks % GRP == 0 and GRP % NBUF == 0
  ngroups = nchunks // GRP

  mesh = plsc.VectorSubcoreMesh(core_axis_name="c", subcore_axis_name="s")
  out_rows = 2 * NPAD if split_edges else nb * NPAD

  scratch = (
      [pltpu.VMEM((GRP * K,), jnp.int32),
       pltpu.VMEM((GRP, 1, K), jnp.int32)]
      + [pltpu.VMEM((K, fb), jnp.float32) for _ in range(NBUF)]
      + [pltpu.SemaphoreType.DMA for _ in range(2 * NBUF)]
      + [pltpu.VMEM_SHARED((NPAD, fb), jnp.float32)]
  )

  @functools.partial(
      pl.kernel, mesh=mesh,
      out_type=jax.ShapeDtypeStruct((out_rows, fb), jnp.float32),
      scratch_types=scratch,
  )
  def k(h_hbm, srcx_hbm, dstx_hbm, zeros_hbm, ones_hbm, out_hbm,
        sidx_g, didx_g, *bufs_sems_acc):
    rows = list(bufs_sems_acc[:NBUF])
    gsem = list(bufs_sems_acc[NBUF:2 * NBUF])
    ssem = list(bufs_sems_acc[2 * NBUF:3 * NBUF])
    acc = bufs_sems_acc[3 * NBUF]

    c = lax.axis_index("c")
    s = lax.axis_index("s")
    wid = s * NC + c
    ebase = (wid if split_edges else s) * epw
    cbase = (wid if split_edges else s) * nchunks
    r0 = s * ROWS_PER_TILE

    if const_rows:
      pltpu.sync_copy(ones_hbm, rows[0])

    def gather(l, b):
      pltpu.async_copy(h_hbm.at[sidx_g.at[pl.ds(l * K, K)]], rows[b], gsem[b])

    def gwait(b):
      pltpu.make_async_copy(h_hbm.at[sidx_g.at[pl.ds(0, K)]], rows[b],
                            gsem[b]).wait()

    def scatter(l, b):
      pltpu.async_copy(rows[0 if const_rows else b],
                       acc.at[didx_g.at[l, 0]], ssem[b], add=True)

    def swait(b):
      pltpu.make_async_copy(rows[0 if const_rows else b],
                            acc.at[didx_g.at[0, 0]], ssem[b]).wait()

    for kb in range(blocks_per_core):
      bglob = 0 if split_edges else c * blocks_per_core + kb
      pltpu.sync_copy(zeros_hbm, acc.at[pl.ds(r0, ROWS_PER_TILE)])
      plsc.subcore_barrier()

      def group(g, carry):
        pltpu.sync_copy(dstx_hbm.at[pl.ds(cbase + g * GRP, GRP)], didx_g)
        if const_rows:
          def cbody(t, carry2):
            for b in range(NBUF):
              l = t * NBUF + b
              pl.when(t > 0)(functools.partial(swait, b))
              scatter(l, b)
            return carry2

          lax.fori_loop(0, GRP // NBUF, cbody, 0)
        else:
          pltpu.sync_copy(
              srcx_hbm.at[pl.ds(bglob * E + ebase + g * GRP * K, GRP * K)],
              sidx_g)
          for b in range(PRE):
            gather(b, b)

          def body(t, carry2):
            for b in range(NBUF):
              l = t * NBUF + b
              gwait(b)
              scatter(l, b)
              bn = (b + PRE) % NBUF

              def refill(bn=bn, l=l):
                pl.when(l >= NBUF - PRE)(functools.partial(swait, bn))
                gather(l + PRE, bn)

              pl.when(l + PRE < GRP)(refill)
            return carry2

          lax.fori_loop(0, GRP // NBUF, body, 0)
        for b in range(NBUF):
          swait(b)
        return carry

      lax.fori_loop(0, ngroups, group, 0)

      plsc.subcore_barrier()
      obase = (c * NPAD if split_edges else bglob * NPAD) + r0
      pltpu.sync_copy(acc.at[pl.ds(r0, ROWS_PER_TILE)],
                      out_hbm.at[pl.ds(obase, ROWS_PER_TILE)])
      if kb + 1 < blocks_per_core:
        plsc.subcore_barrier()

  return k


_segsum_d = _make_sc_segsum(D // 128, 128)
_segsum_c = _make_sc_segsum(1, 128)
_deg_hist = _make_sc_segsum(1, 128, const_rows=True)


R = 1000
GRID = N // R


def _deg_vec(parts_ref):
  d = parts_ref[0, :, :1] + parts_ref[1, :, :1]
  return d


def _dinv(deg):
  return jnp.where(deg > 0, lax.rsqrt(jnp.maximum(deg, 1e-12)), 0.0)


def _scale1_body(parts_ref, x_ref, out_ref):
  deg = _deg_vec(parts_ref)
  xs = x_ref[...] * _dinv(deg)
  for kk in range(2):
    out_ref[kk] = xs[:, kk * 128:(kk + 1) * 128]


def _scale2_body(parts_ref, y_ref, out_ref):
  deg = _deg_vec(parts_ref)
  dinv2 = jnp.where(deg > 0, 1.0 / jnp.maximum(deg, 1e-12), 0.0)
  for kk in range(2):
    out_ref[kk] = y_ref[kk] * dinv2


def _scale3_body(parts_ref, y_ref, blk_ref, flat_ref):
  deg = _deg_vec(parts_ref)
  di = _dinv(deg)
  hs = [y_ref[kk] * di for kk in range(2)]
  for kk in range(2):
    blk_ref[kk] = hs[kk]
  flat_ref[...] = jnp.concatenate(hs, axis=1)


def _selu(x):
  return _SELU_SCALE * jnp.where(x > 0, x, _SELU_ALPHA * (jnp.exp(x) - 1.0))


def _layer1_body(parts_ref, sh_ref, h_ref, wl_ref, bl_ref, wr_ref, wl2_ref,
                 z1_ref, flat_ref):
  deg = _deg_vec(parts_ref)
  cnt = jnp.maximum(deg, 1.0)
  mean = jnp.concatenate([sh_ref[0], sh_ref[1]], axis=1) / cnt
  h1 = (jnp.dot(mean, wl_ref[...], preferred_element_type=jnp.float32)
        + bl_ref[...]
        + jnp.dot(h_ref[...], wr_ref[...], preferred_element_type=jnp.float32))
  h1 = _selu(h1)
  z1 = jnp.dot(h1, wl2_ref[...], preferred_element_type=jnp.float32)
  z1_ref[...] = jnp.concatenate(
      [z1, jnp.zeros((z1.shape[0], 128 - C), z1.dtype)], axis=1)
  flat_ref[...] = h1


def _softmax(z):
  m = jnp.max(z, axis=1, keepdims=True)
  e = jnp.exp(z - m)
  return e / jnp.sum(e, axis=1, keepdims=True)


def _layer2_body(parts_ref, s1_ref, h1_ref, bl_ref, wr_ref, t_ref,
                 p_ref, q_ref, qs_ref):
  deg = _deg_vec(parts_ref)
  cnt = jnp.maximum(deg, 1.0)
  sz1 = (s1_ref[0] + s1_ref[1])[:, :C]
  h2 = (sz1 / cnt
        + bl_ref[...]
        + jnp.dot(h1_ref[...], wr_ref[...], preferred_element_type=jnp.float32))
  p = _softmax(h2)
  q = jnp.dot(p, t_ref[...], preferred_element_type=jnp.float32)
  p_ref[...] = p
  q_ref[...] = q
  qs_ref[...] = jnp.concatenate(
      [q * _dinv(deg), jnp.zeros((q.shape[0], 128 - C), q.dtype)], axis=1)


def _scale4_body(parts_ref, y_ref, out_ref):
  deg = _deg_vec(parts_ref)
  dinv2 = jnp.where(deg > 0, 1.0 / jnp.maximum(deg, 1e-12), 0.0)
  out_ref[...] = (y_ref[0] + y_ref[1]) * dinv2


def _final_body(parts_ref, y_ref, out_ref):
  deg = _deg_vec(parts_ref)
  y = (y_ref[0] + y_ref[1])[:, :C] * _dinv(deg)
  out_ref[...] = _softmax(y)


def _bs(shape, imap):
  return pl.BlockSpec(shape, imap)


_PARTS_BS = _bs((2, R, 16), lambda i: (0, i, 0))


def _tc_call(body, in_specs, out_specs, out_shapes, *args):
  return pl.pallas_call(
      body, grid=(GRID,), in_specs=in_specs, out_specs=out_specs,
      out_shape=out_shapes)(*args)


def kernel(x, edge_index, T, W_l1, b_l1, W_r1, W_l2, b_l2, W_r2):
  src = edge_index[0]
  dst = edge_index[1]

  z128 = jnp.zeros((ROWS_PER_TILE, 128), jnp.float32)
  ones128 = jnp.ones((40, 128), jnp.float32)
  d40 = jnp.zeros((40, 128), jnp.float32)

  dstx40 = dst.reshape(E // 40, 1, 40)
  srcx1 = src
  off2 = (jnp.arange(2, dtype=jnp.int32) * NPAD)[:, None]
  srcx2 = (src[None, :] + off2).reshape(2 * E)

  deg_parts = _deg_hist(jnp.zeros((8, 128), jnp.float32), srcx1, dstx40,
                        z128, ones128)
  deg_parts = deg_parts.reshape(2, NPAD, 128)[:, :, :16]

  xs_blk = _tc_call(
      _scale1_body,
      [_PARTS_BS, _bs((R, D), lambda i: (i, 0))],
      _bs((2, R, 128), lambda i: (0, i, 0)),
      jax.ShapeDtypeStruct((2, NPAD, 128), jnp.float32),
      deg_parts, x)

  x2_blk = _segsum_d(xs_blk.reshape(2 * NPAD, 128), srcx2, dstx40, z128, d40)
  x2_blk = x2_blk.reshape(2, NPAD, 128)

  x2s_blk = _tc_call(
      _scale2_body,
      [_PARTS_BS, _bs((2, R, 128), lambda i: (0, i, 0))],
      _bs((2, R, 128), lambda i: (0, i, 0)),
      jax.ShapeDtypeStruct((2, NPAD, 128), jnp.float32),
      deg_parts, x2_blk)

  x3_blk = _segsum_d(x2s_blk.reshape(2 * NPAD, 128), srcx2, dstx40, z128, d40)
  x3_blk = x3_blk.reshape(2, NPAD, 128)

  h_blk, h_flat = _tc_call(
      _scale3_body,
      [_PARTS_BS, _bs((2, R, 128), lambda i: (0, i, 0))],
      [_bs((2, R, 128), lambda i: (0, i, 0)), _bs((R, D), lambda i: (i, 0))],
      [jax.ShapeDtypeStruct((2, NPAD, 128), jnp.float32),
       jax.ShapeDtypeStruct((N, D), jnp.float32)],
      deg_parts, x3_blk)

  sh_blk = _segsum_d(h_blk.reshape(2 * NPAD, 128), srcx2, dstx40, z128, d40)
  sh_blk = sh_blk.reshape(2, NPAD, 128)

  b_l1r = b_l1.reshape(1, H)
  z1_pad, h1_flat = _tc_call(
      _layer1_body,
      [_PARTS_BS,
       _bs((2, R, 128), lambda i: (0, i, 0)),
       _bs((R, D), lambda i: (i, 0)),
       _bs((D, H), lambda i: (0, 0)),
       _bs((1, H), lambda i: (0, 0)),
       _bs((D, H), lambda i: (0, 0)),
       _bs((H, C), lambda i: (0, 0))],
      [_bs((R, 128), lambda i: (i, 0)), _bs((R, H), lambda i: (i, 0))],
      [jax.ShapeDtypeStruct((N, 128), jnp.float32),
       jax.ShapeDtypeStruct((N, H), jnp.float32)],
      deg_parts, sh_blk, h_flat, W_l1, b_l1r, W_r1, W_l2)

  s1_blk = _segsum_c(z1_pad, srcx1, dstx40, z128, d40).reshape(2, NPAD, 128)

  b_l2r = b_l2.reshape(1, C)
  p, q, qs = _tc_call(
      _layer2_body,
      [_PARTS_BS,
       _bs((2, R, 128), lambda i: (0, i, 0)),
       _bs((R, H), lambda i: (i, 0)),
       _bs((1, C), lambda i: (0, 0)),
       _bs((H, C), lambda i: (0, 0)),
       _bs((C, C), lambda i: (0, 0))],
      [_bs((R, C), lambda i: (i, 0))] * 2 + [_bs((R, 128), lambda i: (i, 0))],
      [jax.ShapeDtypeStruct((N, C), jnp.float32)] * 2
      + [jax.ShapeDtypeStruct((N, 128), jnp.float32)],
      deg_parts, s1_blk, h1_flat, b_l2r, W_r2, T)

  y1_parts = _segsum_c(qs, srcx1, dstx40, z128,
                       d40).reshape(2, NPAD, 128)

  y1s = _tc_call(
      _scale4_body,
      [_PARTS_BS, _bs((2, R, 128), lambda i: (0, i, 0))],
      _bs((R, 128), lambda i: (i, 0)),
      jax.ShapeDtypeStruct((N, 128), jnp.float32),
      deg_parts, y1_parts)

  y2_parts = _segsum_c(y1s, srcx1, dstx40, z128,
                       d40).reshape(2, NPAD, 128)

  p_yt = _tc_call(
      _final_body,
      [_PARTS_BS, _bs((2, R, 128), lambda i: (0, i, 0))],
      _bs((R, C), lambda i: (i, 0)),
      jax.ShapeDtypeStruct((N, C), jnp.float32),
      deg_parts, y2_parts)

  return (p, q, p_yt)


# speedup vs baseline: 11.6447x; 1.0550x over previous
import functools

import jax
import jax.numpy as jnp
from jax import lax
from jax.experimental import pallas as pl
from jax.experimental.pallas import tpu as pltpu
from jax.experimental.pallas import tpu_sc as plsc

N = 10000
E = 160000
D = 256
H = 512
C = 64

NC = 2
NS = 16
NPAD = 10240
ROWS_PER_TILE = NPAD // NS

_SELU_ALPHA = 1.6732632423543772
_SELU_SCALE = 1.0507009873554805


NBUF = 5
PRE = 4


def _make_sc_segsum(nb, fb, const_rows=False):
  split_edges = (nb == 1)
  blocks_per_core = 1 if split_edges else nb // 2
  epw = E // (NC * NS) if split_edges else E // NS
  K = 40
  nchunks = epw // K
  GRP = 50 if not split_edges else 25
  assert nchunks % GRP == 0 and GRP % NBUF == 0
  ngroups = nchunks // GRP

  mesh = plsc.VectorSubcoreMesh(core_axis_name="c", subcore_axis_name="s")
  out_rows = 2 * NPAD if split_edges else nb * NPAD

  scratch = (
      [pltpu.VMEM((GRP * K,), jnp.int32),
       pltpu.VMEM((GRP, 1, K), jnp.int32)]
      + [pltpu.VMEM((K, fb), jnp.float32) for _ in range(NBUF)]
      + [pltpu.SemaphoreType.DMA for _ in range(2 * NBUF)]
      + [pltpu.VMEM_SHARED((NPAD, fb), jnp.float32)]
  )

  @functools.partial(
      pl.kernel, mesh=mesh,
      out_type=jax.ShapeDtypeStruct((out_rows, fb), jnp.float32),
      scratch_types=scratch,
  )
  def k(h_hbm, srcx_hbm, dstx_hbm, zeros_hbm, ones_hbm, out_hbm,
        sidx_g, didx_g, *bufs_sems_acc):
    rows = list(bufs_sems_acc[:NBUF])
    gsem = list(bufs_sems_acc[NBUF:2 * NBUF])
    ssem = list(bufs_sems_acc[2 * NBUF:3 * NBUF])
    acc = bufs_sems_acc[3 * NBUF]

    c = lax.axis_index("c")
    s = lax.axis_index("s")
    wid = s * NC + c
    ebase = (wid if split_edges else s) * epw
    cbase = (wid if split_edges else s) * nchunks
    r0 = s * ROWS_PER_TILE

    if const_rows:
      pltpu.sync_copy(ones_hbm, rows[0])

    def gather(l, b):
      pltpu.async_copy(h_hbm.at[sidx_g.at[pl.ds(l * K, K)]], rows[b], gsem[b])

    def gwait(b):
      pltpu.make_async_copy(h_hbm.at[sidx_g.at[pl.ds(0, K)]], rows[b],
                            gsem[b]).wait()

    def scatter(l, b):
      pltpu.async_copy(rows[0 if const_rows else b],
                       acc.at[didx_g.at[l, 0]], ssem[b], add=True)

    def swait(b):
      pltpu.make_async_copy(rows[0 if const_rows else b],
                            acc.at[didx_g.at[0, 0]], ssem[b]).wait()

    for kb in range(blocks_per_core):
      bglob = 0 if split_edges else c * blocks_per_core + kb
      pltpu.sync_copy(zeros_hbm, acc.at[pl.ds(r0, ROWS_PER_TILE)])
      plsc.subcore_barrier()

      def group(g, carry):
        pltpu.sync_copy(dstx_hbm.at[pl.ds(cbase + g * GRP, GRP)], didx_g)
        if const_rows:
          def cbody(t, carry2):
            for b in range(NBUF):
              l = t * NBUF + b
              pl.when(t > 0)(functools.partial(swait, b))
              scatter(l, b)
            return carry2

          lax.fori_loop(0, GRP // NBUF, cbody, 0)
        else:
          pltpu.sync_copy(
              srcx_hbm.at[pl.ds(bglob * E + ebase + g * GRP * K, GRP * K)],
              sidx_g)
          for b in range(PRE):
            gather(b, b)

          def body(t, carry2):
            for b in range(NBUF):
              l = t * NBUF + b
              gwait(b)
              scatter(l, b)
              bn = (b + PRE) % NBUF

              def refill(bn=bn, l=l):
                pl.when(l >= NBUF - PRE)(functools.partial(swait, bn))
                gather(l + PRE, bn)

              pl.when(l + PRE < GRP)(refill)
            return carry2

          lax.fori_loop(0, GRP // NBUF, body, 0)
        for b in range(NBUF):
          swait(b)
        return carry

      lax.fori_loop(0, ngroups, group, 0)

      plsc.subcore_barrier()
      obase = (c * NPAD if split_edges else bglob * NPAD) + r0
      pltpu.sync_copy(acc.at[pl.ds(r0, ROWS_PER_TILE)],
                      out_hbm.at[pl.ds(obase, ROWS_PER_TILE)])
      if kb + 1 < blocks_per_core:
        plsc.subcore_barrier()

  return k


_segsum_d = _make_sc_segsum(D // 128, 128)
_segsum_c = _make_sc_segsum(1, 128)
_deg_hist = _make_sc_segsum(1, 128, const_rows=True)


R = 1000
GRID = N // R


def _deg_vec(parts_ref):
  d = parts_ref[0, :, :1] + parts_ref[1, :, :1]
  return d


def _dinv(deg):
  return jnp.where(deg > 0, lax.rsqrt(jnp.maximum(deg, 1e-12)), 0.0)


def _scale1_body(parts_ref, x_ref, out_ref):
  deg = _deg_vec(parts_ref)
  xs = x_ref[...] * _dinv(deg)
  for kk in range(2):
    out_ref[kk] = xs[:, kk * 128:(kk + 1) * 128]


def _scale2_body(parts_ref, y_ref, out_ref):
  deg = _deg_vec(parts_ref)
  dinv2 = jnp.where(deg > 0, 1.0 / jnp.maximum(deg, 1e-12), 0.0)
  for kk in range(2):
    out_ref[kk] = y_ref[kk] * dinv2


def _scale3_body(parts_ref, y_ref, blk_ref, flat_ref):
  deg = _deg_vec(parts_ref)
  di = _dinv(deg)
  hs = [y_ref[kk] * di for kk in range(2)]
  for kk in range(2):
    blk_ref[kk] = hs[kk]
  flat_ref[...] = jnp.concatenate(hs, axis=1)


def _selu(x):
  return _SELU_SCALE * jnp.where(x > 0, x, _SELU_ALPHA * (jnp.exp(x) - 1.0))


def _layer1_body(parts_ref, sh_ref, h_ref, wl_ref, bl_ref, wr_ref, wl2_ref,
                 z1_ref, flat_ref):
  deg = _deg_vec(parts_ref)
  cnt = jnp.maximum(deg, 1.0)
  mean = jnp.concatenate([sh_ref[0], sh_ref[1]], axis=1) / cnt
  h1 = (jnp.dot(mean, wl_ref[...], preferred_element_type=jnp.float32)
        + bl_ref[...]
        + jnp.dot(h_ref[...], wr_ref[...], preferred_element_type=jnp.float32))
  h1 = _selu(h1)
  z1 = jnp.dot(h1, wl2_ref[...], preferred_element_type=jnp.float32)
  z1_ref[...] = jnp.concatenate(
      [z1, jnp.zeros((z1.shape[0], 128 - C), z1.dtype)], axis=1)
  flat_ref[...] = h1


def _softmax(z):
  m = jnp.max(z, axis=1, keepdims=True)
  e = jnp.exp(z - m)
  return e / jnp.sum(e, axis=1, keepdims=True)


def _layer2_body(parts_ref, s1_ref, h1_ref, bl_ref, wr_ref, t_ref,
                 p_ref, q_ref, qs_ref):
  deg = _deg_vec(parts_ref)
  cnt = jnp.maximum(deg, 1.0)
  sz1 = (s1_ref[0] + s1_ref[1])[:, :C]
  h2 = (sz1 / cnt
        + bl_ref[...]
        + jnp.dot(h1_ref[...], wr_ref[...], preferred_element_type=jnp.float32))
  p = _softmax(h2)
  q = jnp.dot(p, t_ref[...], preferred_element_type=jnp.float32)
  p_ref[...] = p
  q_ref[...] = q
  qs_ref[...] = jnp.concatenate(
      [q * _dinv(deg), jnp.zeros((q.shape[0], 128 - C), q.dtype)], axis=1)


def _scale4_body(parts_ref, y_ref, out_ref):
  deg = _deg_vec(parts_ref)
  dinv2 = jnp.where(deg > 0, 1.0 / jnp.maximum(deg, 1e-12), 0.0)
  out_ref[...] = (y_ref[0] + y_ref[1]) * dinv2


def _final_body(parts_ref, y_ref, out_ref):
  deg = _deg_vec(parts_ref)
  y = (y_ref[0] + y_ref[1])[:, :C] * _dinv(deg)
  out_ref[...] = _softmax(y)


def _bs(shape, imap):
  return pl.BlockSpec(shape, imap)


_PARTS_BS = _bs((2, R, 16), lambda i: (0, i, 0))


def _tc_call(body, in_specs, out_specs, out_shapes, *args):
  return pl.pallas_call(
      body, grid=(GRID,), in_specs=in_specs, out_specs=out_specs,
      out_shape=out_shapes)(*args)


def kernel(x, edge_index, T, W_l1, b_l1, W_r1, W_l2, b_l2, W_r2):
  src = edge_index[0]
  dst = edge_index[1]

  z128 = jnp.zeros((ROWS_PER_TILE, 128), jnp.float32)
  ones128 = jnp.ones((40, 128), jnp.float32)
  d40 = jnp.zeros((40, 128), jnp.float32)

  dstx40 = dst.reshape(E // 40, 1, 40)
  srcx1 = src
  off2 = (jnp.arange(2, dtype=jnp.int32) * NPAD)[:, None]
  srcx2 = (src[None, :] + off2).reshape(2 * E)

  deg_parts = _deg_hist(jnp.zeros((8, 128), jnp.float32), srcx1, dstx40,
                        z128, ones128)
  deg_parts = deg_parts.reshape(2, NPAD, 128)[:, :, :16]

  xs_blk = _tc_call(
      _scale1_body,
      [_PARTS_BS, _bs((R, D), lambda i: (i, 0))],
      _bs((2, R, 128), lambda i: (0, i, 0)),
      jax.ShapeDtypeStruct((2, NPAD, 128), jnp.float32),
      deg_parts, x)

  x2_blk = _segsum_d(xs_blk.reshape(2 * NPAD, 128), srcx2, dstx40, z128, d40)
  x2_blk = x2_blk.reshape(2, NPAD, 128)

  x2s_blk = _tc_call(
      _scale2_body,
      [_PARTS_BS, _bs((2, R, 128), lambda i: (0, i, 0))],
      _bs((2, R, 128), lambda i: (0, i, 0)),
      jax.ShapeDtypeStruct((2, NPAD, 128), jnp.float32),
      deg_parts, x2_blk)

  x3_blk = _segsum_d(x2s_blk.reshape(2 * NPAD, 128), srcx2, dstx40, z128, d40)
  x3_blk = x3_blk.reshape(2, NPAD, 128)

  h_blk, h_flat = _tc_call(
      _scale3_body,
      [_PARTS_BS, _bs((2, R, 128), lambda i: (0, i, 0))],
      [_bs((2, R, 128), lambda i: (0, i, 0)), _bs((R, D), lambda i: (i, 0))],
      [jax.ShapeDtypeStruct((2, NPAD, 128), jnp.float32),
       jax.ShapeDtypeStruct((N, D), jnp.float32)],
      deg_parts, x3_blk)

  sh_blk = _segsum_d(h_blk.reshape(2 * NPAD, 128), srcx2, dstx40, z128, d40)
  sh_blk = sh_blk.reshape(2, NPAD, 128)

  b_l1r = b_l1.reshape(1, H)
  z1_pad, h1_flat = _tc_call(
      _layer1_body,
      [_PARTS_BS,
       _bs((2, R, 128), lambda i: (0, i, 0)),
       _bs((R, D), lambda i: (i, 0)),
       _bs((D, H), lambda i: (0, 0)),
       _bs((1, H), lambda i: (0, 0)),
       _bs((D, H), lambda i: (0, 0)),
       _bs((H, C), lambda i: (0, 0))],
      [_bs((R, 128), lambda i: (i, 0)), _bs((R, H), lambda i: (i, 0))],
      [jax.ShapeDtypeStruct((N, 128), jnp.float32),
       jax.ShapeDtypeStruct((N, H), jnp.float32)],
      deg_parts, sh_blk, h_flat, W_l1, b_l1r, W_r1, W_l2)

  s1_blk = _segsum_c(z1_pad, srcx1, dstx40, z128, d40).reshape(2, NPAD, 128)

  b_l2r = b_l2.reshape(1, C)
  p, q, qs = _tc_call(
      _layer2_body,
      [_PARTS_BS,
       _bs((2, R, 128), lambda i: (0, i, 0)),
       _bs((R, H), lambda i: (i, 0)),
       _bs((1, C), lambda i: (0, 0)),
       _bs((H, C), lambda i: (0, 0)),
       _bs((C, C), lambda i: (0, 0))],
      [_bs((R, C), lambda i: (i, 0))] * 2 + [_bs((R, 128), lambda i: (i, 0))],
      [jax.ShapeDtypeStruct((N, C), jnp.float32)] * 2
      + [jax.ShapeDtypeStruct((N, 128), jnp.float32)],
      deg_parts, s1_blk, h1_flat, b_l2r, W_r2, T)

  y1_parts = _segsum_c(qs, srcx1, dstx40, z128,
                       d40).reshape(2, NPAD, 128)

  y1s = _tc_call(
      _scale4_body,
      [_PARTS_BS, _bs((2, R, 128), lambda i: (0, i, 0))],
      _bs((R, 128), lambda i: (i, 0)),
      jax.ShapeDtypeStruct((N, 128), jnp.float32),
      deg_parts, y1_parts)

  y2_parts = _segsum_c(y1s, srcx1, dstx40, z128,
                       d40).reshape(2, NPAD, 128)

  p_yt = _tc_call(
      _final_body,
      [_PARTS_BS, _bs((2, R, 128), lambda i: (0, i, 0))],
      _bs((R, C), lambda i: (i, 0)),
      jax.ShapeDtypeStruct((N, C), jnp.float32),
      deg_parts, y2_parts)

  return (p, q, p_yt)


# speedup vs baseline: 12.4803x; 1.0718x over previous
import functools

import jax
import jax.numpy as jnp
from jax import lax
from jax.experimental import pallas as pl
from jax.experimental.pallas import tpu as pltpu
from jax.experimental.pallas import tpu_sc as plsc

N = 10000
E = 160000
D = 256
H = 512
C = 64

NC = 2
NS = 16
NPAD = 10240
ROWS_PER_TILE = NPAD // NS

_SELU_ALPHA = 1.6732632423543772
_SELU_SCALE = 1.0507009873554805


NBUF = 5
PRE = 4


def _make_sc_segsum(nb, fb, const_rows=False):
  split_edges = (nb == 1)
  blocks_per_core = 1 if split_edges else nb // 2
  epw = E // (NC * NS) if split_edges else E // NS
  K = 40
  nchunks = epw // K
  GRP = 125
  assert nchunks % GRP == 0 and GRP % NBUF == 0
  ngroups = nchunks // GRP

  mesh = plsc.VectorSubcoreMesh(core_axis_name="c", subcore_axis_name="s")
  out_rows = 2 * NPAD if split_edges else nb * NPAD

  scratch = (
      [pltpu.VMEM((GRP * K,), jnp.int32),
       pltpu.VMEM((GRP, 1, K), jnp.int32)]
      + [pltpu.VMEM((K, fb), jnp.float32) for _ in range(NBUF)]
      + [pltpu.SemaphoreType.DMA for _ in range(2 * NBUF)]
      + [pltpu.VMEM_SHARED((NPAD, fb), jnp.float32)]
  )

  @functools.partial(
      pl.kernel, mesh=mesh,
      out_type=jax.ShapeDtypeStruct((out_rows, fb), jnp.float32),
      scratch_types=scratch,
  )
  def k(h_hbm, srcx_hbm, dstx_hbm, zeros_hbm, ones_hbm, out_hbm,
        sidx_g, didx_g, *bufs_sems_acc):
    rows = list(bufs_sems_acc[:NBUF])
    gsem = list(bufs_sems_acc[NBUF:2 * NBUF])
    ssem = list(bufs_sems_acc[2 * NBUF:3 * NBUF])
    acc = bufs_sems_acc[3 * NBUF]

    c = lax.axis_index("c")
    s = lax.axis_index("s")
    wid = s * NC + c
    ebase = (wid if split_edges else s) * epw
    cbase = (wid if split_edges else s) * nchunks
    r0 = s * ROWS_PER_TILE

    if const_rows:
      pltpu.sync_copy(ones_hbm, rows[0])

    def gather(l, b):
      pltpu.async_copy(h_hbm.at[sidx_g.at[pl.ds(l * K, K)]], rows[b], gsem[b])

    def gwait(b):
      pltpu.make_async_copy(h_hbm.at[sidx_g.at[pl.ds(0, K)]], rows[b],
                            gsem[b]).wait()

    def scatter(l, b):
      pltpu.async_copy(rows[0 if const_rows else b],
                       acc.at[didx_g.at[l, 0]], ssem[b], add=True)

    def swait(b):
      pltpu.make_async_copy(rows[0 if const_rows else b],
                            acc.at[didx_g.at[0, 0]], ssem[b]).wait()

    for kb in range(blocks_per_core):
      bglob = 0 if split_edges else c * blocks_per_core + kb
      pltpu.sync_copy(zeros_hbm, acc.at[pl.ds(r0, ROWS_PER_TILE)])
      plsc.subcore_barrier()

      def group(g, carry):
        pltpu.sync_copy(dstx_hbm.at[pl.ds(cbase + g * GRP, GRP)], didx_g)
        if const_rows:
          def cbody(t, carry2):
            for b in range(NBUF):
              l = t * NBUF + b
              pl.when(t > 0)(functools.partial(swait, b))
              scatter(l, b)
            return carry2

          lax.fori_loop(0, GRP // NBUF, cbody, 0)
        else:
          pltpu.sync_copy(
              srcx_hbm.at[pl.ds(bglob * E + ebase + g * GRP * K, GRP * K)],
              sidx_g)
          for b in range(PRE):
            gather(b, b)

          def body(t, carry2):
            for b in range(NBUF):
              l = t * NBUF + b
              gwait(b)
              scatter(l, b)
              bn = (b + PRE) % NBUF

              def refill(bn=bn, l=l):
                pl.when(l >= NBUF - PRE)(functools.partial(swait, bn))
                gather(l + PRE, bn)

              pl.when(l + PRE < GRP)(refill)
            return carry2

          lax.fori_loop(0, GRP // NBUF, body, 0)
        for b in range(NBUF):
          swait(b)
        return carry

      lax.fori_loop(0, ngroups, group, 0)

      plsc.subcore_barrier()
      obase = (c * NPAD if split_edges else bglob * NPAD) + r0
      pltpu.sync_copy(acc.at[pl.ds(r0, ROWS_PER_TILE)],
                      out_hbm.at[pl.ds(obase, ROWS_PER_TILE)])
      if kb + 1 < blocks_per_core:
        plsc.subcore_barrier()

  return k


_segsum_d = _make_sc_segsum(D // 128, 128)
_segsum_c = _make_sc_segsum(1, 128)
_deg_hist = _make_sc_segsum(1, 128, const_rows=True)


R = 1000
GRID = N // R


def _deg_vec(parts_ref):
  d = parts_ref[0, :, :1] + parts_ref[1, :, :1]
  return d


def _dinv(deg):
  return jnp.where(deg > 0, lax.rsqrt(jnp.maximum(deg, 1e-12)), 0.0)


def _scale1_body(parts_ref, x_ref, out_ref):
  deg = _deg_vec(parts_ref)
  xs = x_ref[...] * _dinv(deg)
  for kk in range(2):
    out_ref[kk] = xs[:, kk * 128:(kk + 1) * 128]


def _scale2_body(parts_ref, y_ref, out_ref):
  deg = _deg_vec(parts_ref)
  dinv2 = jnp.where(deg > 0, 1.0 / jnp.maximum(deg, 1e-12), 0.0)
  for kk in range(2):
    out_ref[kk] = y_ref[kk] * dinv2


def _scale3_body(parts_ref, y_ref, blk_ref, flat_ref):
  deg = _deg_vec(parts_ref)
  di = _dinv(deg)
  hs = [y_ref[kk] * di for kk in range(2)]
  for kk in range(2):
    blk_ref[kk] = hs[kk]
  flat_ref[...] = jnp.concatenate(hs, axis=1)


def _selu(x):
  return _SELU_SCALE * jnp.where(x > 0, x, _SELU_ALPHA * (jnp.exp(x) - 1.0))


def _layer1_body(parts_ref, sh_ref, h_ref, wl_ref, bl_ref, wr_ref, wl2_ref,
                 z1_ref, flat_ref):
  deg = _deg_vec(parts_ref)
  cnt = jnp.maximum(deg, 1.0)
  mean = jnp.concatenate([sh_ref[0], sh_ref[1]], axis=1) / cnt
  h1 = (jnp.dot(mean, wl_ref[...], preferred_element_type=jnp.float32)
        + bl_ref[...]
        + jnp.dot(h_ref[...], wr_ref[...], preferred_element_type=jnp.float32))
  h1 = _selu(h1)
  z1 = jnp.dot(h1, wl2_ref[...], preferred_element_type=jnp.float32)
  z1_ref[...] = jnp.concatenate(
      [z1, jnp.zeros((z1.shape[0], 128 - C), z1.dtype)], axis=1)
  flat_ref[...] = h1


def _softmax(z):
  m = jnp.max(z, axis=1, keepdims=True)
  e = jnp.exp(z - m)
  return e / jnp.sum(e, axis=1, keepdims=True)


def _layer2_body(parts_ref, s1_ref, h1_ref, bl_ref, wr_ref, t_ref,
                 p_ref, q_ref, qs_ref):
  deg = _deg_vec(parts_ref)
  cnt = jnp.maximum(deg, 1.0)
  sz1 = (s1_ref[0] + s1_ref[1])[:, :C]
  h2 = (sz1 / cnt
        + bl_ref[...]
        + jnp.dot(h1_ref[...], wr_ref[...], preferred_element_type=jnp.float32))
  p = _softmax(h2)
  q = jnp.dot(p, t_ref[...], preferred_element_type=jnp.float32)
  p_ref[...] = p
  q_ref[...] = q
  qs_ref[...] = jnp.concatenate(
      [q * _dinv(deg), jnp.zeros((q.shape[0], 128 - C), q.dtype)], axis=1)


def _scale4_body(parts_ref, y_ref, out_ref):
  deg = _deg_vec(parts_ref)
  dinv2 = jnp.where(deg > 0, 1.0 / jnp.maximum(deg, 1e-12), 0.0)
  out_ref[...] = (y_ref[0] + y_ref[1]) * dinv2


def _final_body(parts_ref, y_ref, out_ref):
  deg = _deg_vec(parts_ref)
  y = (y_ref[0] + y_ref[1])[:, :C] * _dinv(deg)
  out_ref[...] = _softmax(y)


def _bs(shape, imap):
  return pl.BlockSpec(shape, imap)


_PARTS_BS = _bs((2, R, 16), lambda i: (0, i, 0))


def _tc_call(body, in_specs, out_specs, out_shapes, *args):
  return pl.pallas_call(
      body, grid=(GRID,), in_specs=in_specs, out_specs=out_specs,
      out_shape=out_shapes)(*args)


def kernel(x, edge_index, T, W_l1, b_l1, W_r1, W_l2, b_l2, W_r2):
  src = edge_index[0]
  dst = edge_index[1]

  z128 = jnp.zeros((ROWS_PER_TILE, 128), jnp.float32)
  ones128 = jnp.ones((40, 128), jnp.float32)
  d40 = jnp.zeros((40, 128), jnp.float32)

  dstx40 = dst.reshape(E // 40, 1, 40)
  srcx1 = src
  off2 = (jnp.arange(2, dtype=jnp.int32) * NPAD)[:, None]
  srcx2 = (src[None, :] + off2).reshape(2 * E)

  deg_parts = _deg_hist(jnp.zeros((8, 128), jnp.float32), srcx1, dstx40,
                        z128, ones128)
  deg_parts = deg_parts.reshape(2, NPAD, 128)[:, :, :16]

  xs_blk = _tc_call(
      _scale1_body,
      [_PARTS_BS, _bs((R, D), lambda i: (i, 0))],
      _bs((2, R, 128), lambda i: (0, i, 0)),
      jax.ShapeDtypeStruct((2, NPAD, 128), jnp.float32),
      deg_parts, x)

  x2_blk = _segsum_d(xs_blk.reshape(2 * NPAD, 128), srcx2, dstx40, z128, d40)
  x2_blk = x2_blk.reshape(2, NPAD, 128)

  x2s_blk = _tc_call(
      _scale2_body,
      [_PARTS_BS, _bs((2, R, 128), lambda i: (0, i, 0))],
      _bs((2, R, 128), lambda i: (0, i, 0)),
      jax.ShapeDtypeStruct((2, NPAD, 128), jnp.float32),
      deg_parts, x2_blk)

  x3_blk = _segsum_d(x2s_blk.reshape(2 * NPAD, 128), srcx2, dstx40, z128, d40)
  x3_blk = x3_blk.reshape(2, NPAD, 128)

  h_blk, h_flat = _tc_call(
      _scale3_body,
      [_PARTS_BS, _bs((2, R, 128), lambda i: (0, i, 0))],
      [_bs((2, R, 128), lambda i: (0, i, 0)), _bs((R, D), lambda i: (i, 0))],
      [jax.ShapeDtypeStruct((2, NPAD, 128), jnp.float32),
       jax.ShapeDtypeStruct((N, D), jnp.float32)],
      deg_parts, x3_blk)

  sh_blk = _segsum_d(h_blk.reshape(2 * NPAD, 128), srcx2, dstx40, z128, d40)
  sh_blk = sh_blk.reshape(2, NPAD, 128)

  b_l1r = b_l1.reshape(1, H)
  z1_pad, h1_flat = _tc_call(
      _layer1_body,
      [_PARTS_BS,
       _bs((2, R, 128), lambda i: (0, i, 0)),
       _bs((R, D), lambda i: (i, 0)),
       _bs((D, H), lambda i: (0, 0)),
       _bs((1, H), lambda i: (0, 0)),
       _bs((D, H), lambda i: (0, 0)),
       _bs((H, C), lambda i: (0, 0))],
      [_bs((R, 128), lambda i: (i, 0)), _bs((R, H), lambda i: (i, 0))],
      [jax.ShapeDtypeStruct((N, 128), jnp.float32),
       jax.ShapeDtypeStruct((N, H), jnp.float32)],
      deg_parts, sh_blk, h_flat, W_l1, b_l1r, W_r1, W_l2)

  s1_blk = _segsum_c(z1_pad, srcx1, dstx40, z128, d40).reshape(2, NPAD, 128)

  b_l2r = b_l2.reshape(1, C)
  p, q, qs = _tc_call(
      _layer2_body,
      [_PARTS_BS,
       _bs((2, R, 128), lambda i: (0, i, 0)),
       _bs((R, H), lambda i: (i, 0)),
       _bs((1, C), lambda i: (0, 0)),
       _bs((H, C), lambda i: (0, 0)),
       _bs((C, C), lambda i: (0, 0))],
      [_bs((R, C), lambda i: (i, 0))] * 2 + [_bs((R, 128), lambda i: (i, 0))],
      [jax.ShapeDtypeStruct((N, C), jnp.float32)] * 2
      + [jax.ShapeDtypeStruct((N, 128), jnp.float32)],
      deg_parts, s1_blk, h1_flat, b_l2r, W_r2, T)

  y1_parts = _segsum_c(qs, srcx1, dstx40, z128,
                       d40).reshape(2, NPAD, 128)

  y1s = _tc_call(
      _scale4_body,
      [_PARTS_BS, _bs((2, R, 128), lambda i: (0, i, 0))],
      _bs((R, 128), lambda i: (i, 0)),
      jax.ShapeDtypeStruct((N, 128), jnp.float32),
      deg_parts, y1_parts)

  y2_parts = _segsum_c(y1s, srcx1, dstx40, z128,
                       d40).reshape(2, NPAD, 128)

  p_yt = _tc_call(
      _final_body,
      [_PARTS_BS, _bs((2, R, 128), lambda i: (0, i, 0))],
      _bs((R, C), lambda i: (i, 0)),
      jax.ShapeDtypeStruct((N, C), jnp.float32),
      deg_parts, y2_parts)

  return (p, q, p_yt)
